# baseline (speedup 1.0000x reference)
import functools

import jax
import jax.numpy as jnp
from jax import lax
from jax.experimental import pallas as pl
from jax.experimental.pallas import tpu as pltpu
from jax.experimental.pallas import tpu_sc as plsc

N = 10000
E = 320000
F = 128
H = 64
G = 64

NC = 2
NS = 16
NW = NC * NS

CHUNK = 128
NCH = 80
PT = NCH * CHUNK
EPAD = PT * NW
NTRASH = 512
NTOT = N + NTRASH
TRASH_PER_TILE = NTRASH // NS

NBUF = 8
LOOKAHEAD = 4

ROWS_PER_TILE = 624
TAIL_OFF = NS * ROWS_PER_TILE
TAIL_ROWS = N - TAIL_OFF
DEGW = 16

_SC_MESH = dict(
    mesh=plsc.VectorSubcoreMesh(core_axis_name="c", subcore_axis_name="s"),
    compiler_params=pltpu.CompilerParams(use_tc_tiling_on_sc=False),
)


def _sc_deg_body(dstp_hbm, ones_hbm, zeros_hbm, out_hbm, dst_t, ones_v, acc_sh, sem_s):
    cid = lax.axis_index("c")
    sid = lax.axis_index("s")
    wid = cid * NS + sid
    r0 = sid * ROWS_PER_TILE
    pltpu.sync_copy(zeros_hbm, acc_sh.at[pl.ds(r0, ROWS_PER_TILE)])

    pltpu.sync_copy(zeros_hbm.at[pl.ds(0, TRASH_PER_TILE)],
                    acc_sh.at[pl.ds(N + sid * TRASH_PER_TILE, TRASH_PER_TILE)])

    @pl.when(sid == 0)
    def _zero_tail():
        pltpu.sync_copy(zeros_hbm.at[pl.ds(0, TAIL_ROWS)],
                        acc_sh.at[pl.ds(TAIL_OFF, TAIL_ROWS)])

    pltpu.sync_copy(ones_hbm, ones_v)
    pltpu.sync_copy(dstp_hbm.at[wid], dst_t)
    plsc.subcore_barrier()

    def _wait_scatter(j):
        pltpu.make_async_copy(ones_v, acc_sh.at[dst_t.at[0]], sem_s.at[j]).wait()

    def body(k, carry):
        for j in range(NBUF):
            p = k * NBUF + j

            @pl.when(p >= NBUF)
            def _ws(j=j):
                _wait_scatter(j)

            pltpu.async_copy(ones_v, acc_sh.at[dst_t.at[p]], sem_s.at[j], add=True)
        return carry

    lax.fori_loop(0, NCH // NBUF, body, 0)
    for j in range(NBUF):
        _wait_scatter(j)
    plsc.subcore_barrier()
    pltpu.sync_copy(acc_sh.at[pl.ds(r0, ROWS_PER_TILE)],
                    out_hbm.at[cid, pl.ds(r0, ROWS_PER_TILE)])

    @pl.when(sid == 0)
    def _write_tail():
        pltpu.sync_copy(acc_sh.at[pl.ds(TAIL_OFF, TAIL_ROWS)],
                        out_hbm.at[cid, pl.ds(TAIL_OFF, TAIL_ROWS)])


_sc_deg = functools.partial(
    pl.kernel,
    out_type=jax.ShapeDtypeStruct((NC, N, DEGW), jnp.float32),
    scratch_types=[
        pltpu.VMEM((NCH, CHUNK), jnp.int32),
        pltpu.VMEM((CHUNK, DEGW), jnp.float32),
        pltpu.VMEM_SHARED((NTOT, DEGW), jnp.float32),
        pltpu.SemaphoreType.DMA((NBUF,)),
    ],
    **_SC_MESH,
)(_sc_deg_body)


def _sc_agg_body(hs_hbm, srcp_hbm, dstp_hbm, zeros_hbm, out_hbm,
                 src_t, dst_t, rows, acc_sh, sem_g, sem_s):
    cid = lax.axis_index("c")
    sid = lax.axis_index("s")
    wid = cid * NS + sid
    r0 = sid * ROWS_PER_TILE
    pltpu.sync_copy(zeros_hbm, acc_sh.at[pl.ds(r0, ROWS_PER_TILE)])

    pltpu.sync_copy(zeros_hbm.at[pl.ds(0, TRASH_PER_TILE)],
                    acc_sh.at[pl.ds(N + sid * TRASH_PER_TILE, TRASH_PER_TILE)])

    @pl.when(sid == 0)
    def _zero_tail():
        pltpu.sync_copy(zeros_hbm.at[pl.ds(0, TAIL_ROWS)],
                        acc_sh.at[pl.ds(TAIL_OFF, TAIL_ROWS)])

    pltpu.sync_copy(srcp_hbm.at[wid], src_t)
    pltpu.sync_copy(dstp_hbm.at[wid], dst_t)
    plsc.subcore_barrier()

    def _gather(c, j):
        pltpu.async_copy(hs_hbm.at[src_t.at[c]], rows.at[j], sem_g.at[j])

    def _wait_gather(c, j):
        pltpu.make_async_copy(hs_hbm.at[src_t.at[c]], rows.at[j], sem_g.at[j]).wait()

    def _scatter(c, j):
        pltpu.async_copy(rows.at[j], acc_sh.at[dst_t.at[c]], sem_s.at[j], add=True)

    def _wait_scatter(j):
        pltpu.make_async_copy(rows.at[j], acc_sh.at[dst_t.at[0]], sem_s.at[j]).wait()

    for j in range(LOOKAHEAD):
        _gather(j, j)

    def body(k, carry):
        for j in range(NBUF):
            p = k * NBUF + j
            pf = p + LOOKAHEAD
            jj = (j + LOOKAHEAD) % NBUF

            @pl.when(pf < NCH)
            def _prefetch(p=p, pf=pf, jj=jj):
                @pl.when(p >= LOOKAHEAD)
                def _ws(jj=jj):
                    _wait_scatter(jj)

                _gather(pf, jj)

            _wait_gather(p, j)
            _scatter(p, j)
        return carry

    lax.fori_loop(0, NCH // NBUF, body, 0)
    for j in range(NBUF):
        _wait_scatter(j)
    plsc.subcore_barrier()
    pltpu.sync_copy(acc_sh.at[pl.ds(r0, ROWS_PER_TILE)],
                    out_hbm.at[cid, pl.ds(r0, ROWS_PER_TILE)])

    @pl.when(sid == 0)
    def _write_tail():
        pltpu.sync_copy(acc_sh.at[pl.ds(TAIL_OFF, TAIL_ROWS)],
                        out_hbm.at[cid, pl.ds(TAIL_OFF, TAIL_ROWS)])


_sc_agg = functools.partial(
    pl.kernel,
    out_type=jax.ShapeDtypeStruct((NC, N, H), jnp.float32),
    scratch_types=[
        pltpu.VMEM((NCH, CHUNK), jnp.int32),
        pltpu.VMEM((NCH, CHUNK), jnp.int32),
        pltpu.VMEM((NBUF, CHUNK, H), jnp.float32),
        pltpu.VMEM_SHARED((NTOT, H), jnp.float32),
        pltpu.SemaphoreType.DMA((NBUF,)),
        pltpu.SemaphoreType.DMA((NBUF,)),
    ],
    **_SC_MESH,
)(_sc_agg_body)


BLK = 2000
NBLK = N // BLK


def _tc_mm1_body(x_ref, w1_ref, u_ref):
    u_ref[...] = jnp.dot(x_ref[...], w1_ref[...], preferred_element_type=jnp.float32)


def _tc_mm1(x, w1):
    return pl.pallas_call(
        _tc_mm1_body,
        grid=(NBLK,),
        in_specs=[
            pl.BlockSpec((BLK, F), lambda i: (i, 0)),
            pl.BlockSpec((F, H), lambda i: (0, 0)),
        ],
        out_specs=pl.BlockSpec((BLK, H), lambda i: (i, 0)),
        out_shape=jax.ShapeDtypeStruct((N, H), jnp.float32),
    )(x, w1)


def _tc1_body(u_ref, degp_ref, hs_ref, dinv_ref):
    deg = degp_ref[0, :, 0:1] + degp_ref[1, :, 0:1] + 1.0
    dinv = lax.rsqrt(deg)
    hs_ref[...] = u_ref[...] * dinv
    dinv_ref[...] = dinv


def _tc1(u, degp):
    return pl.pallas_call(
        _tc1_body,
        grid=(NBLK,),
        in_specs=[
            pl.BlockSpec((BLK, H), lambda i: (i, 0)),
            pl.BlockSpec((NC, BLK, DEGW), lambda i: (0, i, 0)),
        ],
        out_specs=[
            pl.BlockSpec((BLK, H), lambda i: (i, 0)),
            pl.BlockSpec((BLK, 1), lambda i: (i, 0)),
        ],
        out_shape=[
            jax.ShapeDtypeStruct((N, H), jnp.float32),
            jax.ShapeDtypeStruct((N, 1), jnp.float32),
        ],
    )(u, degp)


def _tc2_body(p_ref, hs1_ref, dinv_ref, b1_ref, w2_ref, hs2_ref):
    dinv = dinv_ref[...]
    t = jnp.maximum(dinv * (p_ref[0] + p_ref[1] + hs1_ref[...]) + b1_ref[...], 0.0)
    hs2_ref[...] = dinv * jnp.dot(t, w2_ref[...], preferred_element_type=jnp.float32)


def _tc2(p, hs1, dinv, b1, w2):
    return pl.pallas_call(
        _tc2_body,
        grid=(NBLK,),
        in_specs=[
            pl.BlockSpec((NC, BLK, H), lambda i: (0, i, 0)),
            pl.BlockSpec((BLK, H), lambda i: (i, 0)),
            pl.BlockSpec((BLK, 1), lambda i: (i, 0)),
            pl.BlockSpec((1, H), lambda i: (0, 0)),
            pl.BlockSpec((H, H), lambda i: (0, 0)),
        ],
        out_specs=pl.BlockSpec((BLK, H), lambda i: (i, 0)),
        out_shape=jax.ShapeDtypeStruct((N, H), jnp.float32),
    )(p, hs1, dinv, b1, w2)


def _tc3_body(q_ref, hs2_ref, dinv_ref, b2_ref, bidx_ref, w3_ref, b3_ref,
              out_ref, pooled_ref):
    i = pl.program_id(0)

    @pl.when(i == 0)
    def _init():
        pooled_ref[...] = jnp.full((G, H), -jnp.inf, jnp.float32)

    dinv = dinv_ref[...]
    h2 = jnp.maximum(dinv * (q_ref[0] + q_ref[1] + hs2_ref[...]) + b2_ref[...], 0.0)
    bidx = bidx_ref[...]

    def gbody(g, carry):
        v = jnp.where(bidx == g, h2, -jnp.inf)
        m = jnp.max(v, axis=0, keepdims=True)
        pooled_ref[pl.ds(g, 1), :] = jnp.maximum(pooled_ref[pl.ds(g, 1), :], m)
        return carry

    g_lo = bidx[0, 0]
    g_hi = bidx[BLK - 1, 0]
    lax.fori_loop(g_lo, g_hi + 1, gbody, 0)

    @pl.when(i == NBLK - 1)
    def _fin():
        out_ref[...] = (jnp.dot(pooled_ref[...], w3_ref[...],
                                preferred_element_type=jnp.float32) + b3_ref[...])


def _tc3(q, hs2, dinv, b2, bidx, w3, b3):
    return pl.pallas_call(
        _tc3_body,
        grid=(NBLK,),
        in_specs=[
            pl.BlockSpec((NC, BLK, H), lambda i: (0, i, 0)),
            pl.BlockSpec((BLK, H), lambda i: (i, 0)),
            pl.BlockSpec((BLK, 1), lambda i: (i, 0)),
            pl.BlockSpec((1, H), lambda i: (0, 0)),
            pl.BlockSpec((BLK, 1), lambda i: (i, 0)),
            pl.BlockSpec((H, 1), lambda i: (0, 0)),
            pl.BlockSpec((1, 1), lambda i: (0, 0)),
        ],
        out_specs=pl.BlockSpec((G, 1), lambda i: (0, 0)),
        out_shape=jax.ShapeDtypeStruct((G, 1), jnp.float32),
        scratch_shapes=[pltpu.VMEM((G, H), jnp.float32)],
    )(q, hs2, dinv, b2, bidx, w3, b3)


def kernel(x, edge_index, batch_index, W1, b1, W2, b2, W3, b3):
    src = edge_index[0]
    dst = edge_index[1]
    npad = EPAD - E
    pad_iota = jnp.arange(npad, dtype=jnp.int32)
    srcp = jnp.concatenate(
        [src, pad_iota % N]).reshape(NW, NCH, CHUNK)
    dstp = jnp.concatenate(
        [dst, N + pad_iota % NTRASH]).reshape(NW, NCH, CHUNK)
    zeros_h = jnp.zeros((ROWS_PER_TILE, H), jnp.float32)
    zeros_d = jnp.zeros((ROWS_PER_TILE, DEGW), jnp.float32)
    ones_d = jnp.ones((CHUNK, DEGW), jnp.float32)

    degp = _sc_deg(dstp, ones_d, zeros_d)
    u1 = _tc_mm1(x, W1)
    hs1, dinv = _tc1(u1, degp)
    p = _sc_agg(hs1, srcp, dstp, zeros_h)
    hs2 = _tc2(p, hs1, dinv, b1.reshape(1, H), W2)
    q = _sc_agg(hs2, srcp, dstp, zeros_h)
    return _tc3(q, hs2, dinv, b2.reshape(1, H), batch_index.reshape(N, 1),
                W3, b3.reshape(1, 1))


# speedup vs baseline: 1.0309x; 1.0309x over previous
import functools

import jax
import jax.numpy as jnp
from jax import lax
from jax.experimental import pallas as pl
from jax.experimental.pallas import tpu as pltpu
from jax.experimental.pallas import tpu_sc as plsc

N = 10000
E = 320000
F = 128
H = 64
G = 64

NC = 2
NS = 16
NW = NC * NS

CHUNK = 128
NCH = 80
PT = NCH * CHUNK
EPAD = PT * NW
NTRASH = 512
NTOT = N + NTRASH
TRASH_PER_TILE = NTRASH // NS

NBUF = 8
LOOKAHEAD = 5

ROWS_PER_TILE = 624
TAIL_OFF = NS * ROWS_PER_TILE
TAIL_ROWS = N - TAIL_OFF
DEGW = 16

_SC_MESH = dict(
    mesh=plsc.VectorSubcoreMesh(core_axis_name="c", subcore_axis_name="s"),
    compiler_params=pltpu.CompilerParams(use_tc_tiling_on_sc=False),
)


def _sc_deg_body(dstp_hbm, ones_hbm, zeros_hbm, out_hbm, dst_t, ones_v, acc_sh, sem_s):
    cid = lax.axis_index("c")
    sid = lax.axis_index("s")
    wid = cid * NS + sid
    r0 = sid * ROWS_PER_TILE
    pltpu.sync_copy(zeros_hbm, acc_sh.at[pl.ds(r0, ROWS_PER_TILE)])

    pltpu.sync_copy(zeros_hbm.at[pl.ds(0, TRASH_PER_TILE)],
                    acc_sh.at[pl.ds(N + sid * TRASH_PER_TILE, TRASH_PER_TILE)])

    @pl.when(sid == 0)
    def _zero_tail():
        pltpu.sync_copy(zeros_hbm.at[pl.ds(0, TAIL_ROWS)],
                        acc_sh.at[pl.ds(TAIL_OFF, TAIL_ROWS)])

    pltpu.sync_copy(ones_hbm, ones_v)
    pltpu.sync_copy(dstp_hbm.at[wid], dst_t)
    plsc.subcore_barrier()

    def _wait_scatter(j):
        pltpu.make_async_copy(ones_v, acc_sh.at[dst_t.at[0]], sem_s.at[j]).wait()

    def body(k, carry):
        for j in range(NBUF):
            p = k * NBUF + j

            @pl.when(p >= NBUF)
            def _ws(j=j):
                _wait_scatter(j)

            pltpu.async_copy(ones_v, acc_sh.at[dst_t.at[p]], sem_s.at[j], add=True)
        return carry

    lax.fori_loop(0, NCH // NBUF, body, 0)
    for j in range(NBUF):
        _wait_scatter(j)
    plsc.subcore_barrier()
    pltpu.sync_copy(acc_sh.at[pl.ds(r0, ROWS_PER_TILE)],
                    out_hbm.at[cid, pl.ds(r0, ROWS_PER_TILE)])

    @pl.when(sid == 0)
    def _write_tail():
        pltpu.sync_copy(acc_sh.at[pl.ds(TAIL_OFF, TAIL_ROWS)],
                        out_hbm.at[cid, pl.ds(TAIL_OFF, TAIL_ROWS)])


_sc_deg = functools.partial(
    pl.kernel,
    out_type=jax.ShapeDtypeStruct((NC, N, DEGW), jnp.float32),
    scratch_types=[
        pltpu.VMEM((NCH, CHUNK), jnp.int32),
        pltpu.VMEM((CHUNK, DEGW), jnp.float32),
        pltpu.VMEM_SHARED((NTOT, DEGW), jnp.float32),
        pltpu.SemaphoreType.DMA((NBUF,)),
    ],
    **_SC_MESH,
)(_sc_deg_body)


def _sc_agg_body(hs_hbm, srcp_hbm, dstp_hbm, zeros_hbm, out_hbm,
                 src_t, dst_t, rows, acc_sh, sem_g, sem_s):
    cid = lax.axis_index("c")
    sid = lax.axis_index("s")
    wid = cid * NS + sid
    r0 = sid * ROWS_PER_TILE
    pltpu.sync_copy(zeros_hbm, acc_sh.at[pl.ds(r0, ROWS_PER_TILE)])

    pltpu.sync_copy(zeros_hbm.at[pl.ds(0, TRASH_PER_TILE)],
                    acc_sh.at[pl.ds(N + sid * TRASH_PER_TILE, TRASH_PER_TILE)])

    @pl.when(sid == 0)
    def _zero_tail():
        pltpu.sync_copy(zeros_hbm.at[pl.ds(0, TAIL_ROWS)],
                        acc_sh.at[pl.ds(TAIL_OFF, TAIL_ROWS)])

    pltpu.sync_copy(srcp_hbm.at[wid], src_t)
    pltpu.sync_copy(dstp_hbm.at[wid], dst_t)
    plsc.subcore_barrier()

    def _gather(c, j):
        pltpu.async_copy(hs_hbm.at[src_t.at[c]], rows.at[j], sem_g.at[j])

    def _wait_gather(c, j):
        pltpu.make_async_copy(hs_hbm.at[src_t.at[c]], rows.at[j], sem_g.at[j]).wait()

    def _scatter(c, j):
        pltpu.async_copy(rows.at[j], acc_sh.at[dst_t.at[c]], sem_s.at[j], add=True)

    def _wait_scatter(j):
        pltpu.make_async_copy(rows.at[j], acc_sh.at[dst_t.at[0]], sem_s.at[j]).wait()

    for j in range(LOOKAHEAD):
        _gather(j, j)

    def body(k, carry):
        for j in range(NBUF):
            p = k * NBUF + j
            pf = p + LOOKAHEAD
            jj = (j + LOOKAHEAD) % NBUF

            @pl.when(pf < NCH)
            def _prefetch(pf=pf, jj=jj):
                @pl.when(pf >= NBUF)
                def _ws(jj=jj):
                    _wait_scatter(jj)

                _gather(pf, jj)

            _wait_gather(p, j)
            _scatter(p, j)
        return carry

    lax.fori_loop(0, NCH // NBUF, body, 0)
    for j in range(NBUF):
        _wait_scatter(j)
    plsc.subcore_barrier()
    pltpu.sync_copy(acc_sh.at[pl.ds(r0, ROWS_PER_TILE)],
                    out_hbm.at[cid, pl.ds(r0, ROWS_PER_TILE)])

    @pl.when(sid == 0)
    def _write_tail():
        pltpu.sync_copy(acc_sh.at[pl.ds(TAIL_OFF, TAIL_ROWS)],
                        out_hbm.at[cid, pl.ds(TAIL_OFF, TAIL_ROWS)])


_sc_agg = functools.partial(
    pl.kernel,
    out_type=jax.ShapeDtypeStruct((NC, N, H), jnp.float32),
    scratch_types=[
        pltpu.VMEM((NCH, CHUNK), jnp.int32),
        pltpu.VMEM((NCH, CHUNK), jnp.int32),
        pltpu.VMEM((NBUF, CHUNK, H), jnp.float32),
        pltpu.VMEM_SHARED((NTOT, H), jnp.float32),
        pltpu.SemaphoreType.DMA((NBUF,)),
        pltpu.SemaphoreType.DMA((NBUF,)),
    ],
    **_SC_MESH,
)(_sc_agg_body)


BLK = 2000
NBLK = N // BLK


def _tc1_body(x_ref, w1_ref, degp_ref, hs_ref, dinv_ref):
    deg = degp_ref[0, :, 0:1] + degp_ref[1, :, 0:1] + 1.0
    dinv = lax.rsqrt(deg)
    u = jnp.dot(x_ref[...], w1_ref[...], preferred_element_type=jnp.float32)
    hs_ref[...] = u * dinv
    dinv_ref[...] = dinv


def _tc1(x, w1, degp):
    return pl.pallas_call(
        _tc1_body,
        grid=(NBLK,),
        in_specs=[
            pl.BlockSpec((BLK, F), lambda i: (i, 0)),
            pl.BlockSpec((F, H), lambda i: (0, 0)),
            pl.BlockSpec((NC, BLK, DEGW), lambda i: (0, i, 0)),
        ],
        out_specs=[
            pl.BlockSpec((BLK, H), lambda i: (i, 0)),
            pl.BlockSpec((BLK, 1), lambda i: (i, 0)),
        ],
        out_shape=[
            jax.ShapeDtypeStruct((N, H), jnp.float32),
            jax.ShapeDtypeStruct((N, 1), jnp.float32),
        ],
    )(x, w1, degp)


def _tc2_body(p_ref, hs1_ref, dinv_ref, b1_ref, w2_ref, hs2_ref):
    dinv = dinv_ref[...]
    t = jnp.maximum(dinv * (p_ref[0] + p_ref[1] + hs1_ref[...]) + b1_ref[...], 0.0)
    hs2_ref[...] = dinv * jnp.dot(t, w2_ref[...], preferred_element_type=jnp.float32)


def _tc2(p, hs1, dinv, b1, w2):
    return pl.pallas_call(
        _tc2_body,
        grid=(NBLK,),
        in_specs=[
            pl.BlockSpec((NC, BLK, H), lambda i: (0, i, 0)),
            pl.BlockSpec((BLK, H), lambda i: (i, 0)),
            pl.BlockSpec((BLK, 1), lambda i: (i, 0)),
            pl.BlockSpec((1, H), lambda i: (0, 0)),
            pl.BlockSpec((H, H), lambda i: (0, 0)),
        ],
        out_specs=pl.BlockSpec((BLK, H), lambda i: (i, 0)),
        out_shape=jax.ShapeDtypeStruct((N, H), jnp.float32),
    )(p, hs1, dinv, b1, w2)


def _tc3_body(q_ref, hs2_ref, dinv_ref, b2_ref, bidx_ref, w3_ref, b3_ref,
              out_ref, pooled_ref):
    i = pl.program_id(0)

    @pl.when(i == 0)
    def _init():
        pooled_ref[...] = jnp.full((G, H), -jnp.inf, jnp.float32)

    dinv = dinv_ref[...]
    h2 = jnp.maximum(dinv * (q_ref[0] + q_ref[1] + hs2_ref[...]) + b2_ref[...], 0.0)
    bidx = bidx_ref[...]

    def gbody(g, carry):
        v = jnp.where(bidx == g, h2, -jnp.inf)
        m = jnp.max(v, axis=0, keepdims=True)
        pooled_ref[pl.ds(g, 1), :] = jnp.maximum(pooled_ref[pl.ds(g, 1), :], m)
        return carry

    g_lo = bidx[0, 0]
    g_hi = bidx[BLK - 1, 0]
    lax.fori_loop(g_lo, g_hi + 1, gbody, 0)

    @pl.when(i == NBLK - 1)
    def _fin():
        out_ref[...] = (jnp.dot(pooled_ref[...], w3_ref[...],
                                preferred_element_type=jnp.float32) + b3_ref[...])


def _tc3(q, hs2, dinv, b2, bidx, w3, b3):
    return pl.pallas_call(
        _tc3_body,
        grid=(NBLK,),
        in_specs=[
            pl.BlockSpec((NC, BLK, H), lambda i: (0, i, 0)),
            pl.BlockSpec((BLK, H), lambda i: (i, 0)),
            pl.BlockSpec((BLK, 1), lambda i: (i, 0)),
            pl.BlockSpec((1, H), lambda i: (0, 0)),
            pl.BlockSpec((BLK, 1), lambda i: (i, 0)),
            pl.BlockSpec((H, 1), lambda i: (0, 0)),
            pl.BlockSpec((1, 1), lambda i: (0, 0)),
        ],
        out_specs=pl.BlockSpec((G, 1), lambda i: (0, 0)),
        out_shape=jax.ShapeDtypeStruct((G, 1), jnp.float32),
        scratch_shapes=[pltpu.VMEM((G, H), jnp.float32)],
    )(q, hs2, dinv, b2, bidx, w3, b3)


def kernel(x, edge_index, batch_index, W1, b1, W2, b2, W3, b3):
    src = edge_index[0]
    dst = edge_index[1]
    npad = EPAD - E
    pad_iota = jnp.arange(npad, dtype=jnp.int32)
    srcp = jnp.concatenate(
        [src, pad_iota % N]).reshape(NW, NCH, CHUNK)
    dstp = jnp.concatenate(
        [dst, N + pad_iota % NTRASH]).reshape(NW, NCH, CHUNK)
    zeros_h = jnp.zeros((ROWS_PER_TILE, H), jnp.float32)
    zeros_d = jnp.zeros((ROWS_PER_TILE, DEGW), jnp.float32)
    ones_d = jnp.ones((CHUNK, DEGW), jnp.float32)

    degp = _sc_deg(dstp, ones_d, zeros_d)
    hs1, dinv = _tc1(x, W1, degp)
    p = _sc_agg(hs1, srcp, dstp, zeros_h)
    hs2 = _tc2(p, hs1, dinv, b1.reshape(1, H), W2)
    q = _sc_agg(hs2, srcp, dstp, zeros_h)
    return _tc3(q, hs2, dinv, b2.reshape(1, H), batch_index.reshape(N, 1),
                W3, b3.reshape(1, 1))
